# baseline (speedup 1.0000x reference)
import functools

import numpy as np

import jax
import jax.numpy as jnp
from jax import lax
from jax.experimental import pallas as pl
from jax.experimental.pallas import tpu as pltpu
from jax.experimental.pallas import tpu_sc as plsc

N_NODES = 100000
D_IN = 128
D_OUT = 384
D3 = D_OUT // 3
B = 16384
K = 16
DW = D_IN // 2

NC = 2
NS = 16
NW = NC * NS
RPW = B // NW
CH = 8
NCH = RPW // CH
NT = 2 * NCH
SELF_CH = 128
NSC = RPW // SELF_CH
VT = NT + NSC
NBUF = 6

_MASK_HI = -65536

_PERM = np.empty(D_IN, np.int32)
for _g in range(D_IN // 32):
    for _k in range(16):
        _PERM[_g * 32 + _k] = _g * 32 + 2 * _k
        _PERM[_g * 32 + 16 + _k] = _g * 32 + 2 * _k + 1


def _sc_body(nodes2_hbm, adj2_hbm, dis2_hbm, tpk_hbm,
             selfpk_out, adj_out, dis_out,
             idx_adj, idx_dis, idx_self,
             rows0, rows1, rows2, rows3, rows4, rows5, ob0, ob1,
             semg0, semg1, semg2, semg3, semg4, semg5,
             semw0, semw1, semself):
    wid = lax.axis_index("s") * NC + lax.axis_index("c")
    base = wid * RPW

    rows = (rows0, rows1, rows2, rows3, rows4, rows5)
    semg = (semg0, semg1, semg2, semg3, semg4, semg5)
    ob = (ob0, ob1)
    semw = (semw0, semw1)

    pltpu.sync_copy(adj2_hbm.at[pl.ds(wid * NCH, NCH), :], idx_adj)
    pltpu.sync_copy(dis2_hbm.at[pl.ds(wid * NCH, NCH), :], idx_dis)
    pltpu.sync_copy(nodes2_hbm.at[pl.ds(wid * NSC, NSC), :], idx_self)

    def fire(t, slot):
        @pl.when(t < NCH)
        def _():
            pltpu.async_copy(tpk_hbm.at[idx_adj.at[t]], rows[slot], semg[slot])

        @pl.when(jnp.logical_and(t >= NCH, t < NT))
        def _():
            pltpu.async_copy(tpk_hbm.at[idx_dis.at[t - NCH]], rows[slot],
                             semg[slot])

        @pl.when(jnp.logical_and(t >= NT, t < VT))
        def _():
            pltpu.async_copy(tpk_hbm.at[idx_self.at[t - NT]], rows[slot],
                             semg[slot])

    def reduce_chunk(slot, oslot):
        def red_node(j, carry):
            rb = j * K
            for g in range(DW // 16):
                col = g * 16
                w = rows[slot][rb, pl.ds(col, 16)]
                acc_e = plsc.bitcast(w << 16, jnp.float32)
                acc_o = plsc.bitcast(w & _MASK_HI, jnp.float32)
                for i in range(1, K):
                    w = rows[slot][rb + i, pl.ds(col, 16)]
                    acc_e = acc_e + plsc.bitcast(w << 16, jnp.float32)
                    acc_o = acc_o + plsc.bitcast(w & _MASK_HI, jnp.float32)
                ob[oslot][j, pl.ds(2 * col, 16)] = acc_e
                ob[oslot][j, pl.ds(2 * col + 16, 16)] = acc_o
            return carry

        lax.fori_loop(0, CH, red_node, 0)

    for s in range(NBUF - 1):
        fire(s, s)

    @pl.loop(0, VT, step=NBUF)
    def outer(t0):
        for b in range(NBUF):
            t = t0 + b
            fire(t + NBUF - 1, (b + NBUF - 1) % NBUF)
            pltpu.make_async_copy(tpk_hbm.at[idx_adj.at[0]], rows[b],
                                  semg[b]).wait()

            @pl.when(t < NT)
            def _():
                @pl.when(t >= 2)
                def _():
                    pltpu.make_async_copy(ob[b % 2], adj_out.at[pl.ds(0, CH)],
                                          semw[b % 2]).wait()

                reduce_chunk(b, b % 2)

                @pl.when(t < NCH)
                def _():
                    pltpu.async_copy(ob[b % 2],
                                     adj_out.at[pl.ds(base + t * CH, CH)],
                                     semw[b % 2])

                @pl.when(t >= NCH)
                def _():
                    pltpu.async_copy(
                        ob[b % 2],
                        dis_out.at[pl.ds(base + (t - NCH) * CH, CH)],
                        semw[b % 2])

            @pl.when(jnp.logical_and(t >= NT, t < VT))
            def _():
                off = base + (t - NT) * SELF_CH
                pltpu.async_copy(rows[b], selfpk_out.at[pl.ds(off, SELF_CH)],
                                 semself)

    for i in range(2):
        t = NT - 2 + i
        pltpu.make_async_copy(ob[t % 2],
                              dis_out.at[pl.ds(base + (t - NCH) * CH, CH)],
                              semw[t % 2]).wait()
    for c in range(NSC):
        pltpu.make_async_copy(
            rows[0], selfpk_out.at[pl.ds(base + c * SELF_CH, SELF_CH)],
            semself).wait()


_sc_gather = functools.partial(
    pl.kernel,
    out_type=[
        jax.ShapeDtypeStruct((B, DW), jnp.int32),
        jax.ShapeDtypeStruct((B, D_IN), jnp.float32),
        jax.ShapeDtypeStruct((B, D_IN), jnp.float32),
    ],
    mesh=plsc.VectorSubcoreMesh(core_axis_name="c", subcore_axis_name="s"),
    compiler_params=pltpu.CompilerParams(needs_layout_passes=False,
                                         use_tc_tiling_on_sc=False),
    scratch_types=(
        [pltpu.VMEM((NCH, CH * K), jnp.int32)] * 2
        + [pltpu.VMEM((NSC, SELF_CH), jnp.int32)]
        + [pltpu.VMEM((CH * K, DW), jnp.int32)] * NBUF
        + [pltpu.VMEM((CH, D_IN), jnp.float32)] * 2
        + [pltpu.SemaphoreType.DMA] * (NBUF + 3)
    ),
)(_sc_body)


def _compose_body(waa_tp, baa, wad_tp, bad, ws_t, wa_t, wd_t, wc_t, bwc,
                  a_ref, c_ref):
    hp = jax.lax.Precision.HIGHEST
    m1 = jnp.dot(wa_t[...], wc_t[D3:2 * D3, :], precision=hp)
    m2 = jnp.dot(wd_t[...], wc_t[2 * D3:D_OUT, :], precision=hp)
    a_ref[0:D_IN, :] = jnp.dot(ws_t[...], wc_t[0:D3, :], precision=hp)
    a_ref[D_IN:2 * D_IN, :] = jnp.dot(waa_tp[...], m1,
                                      precision=hp) * (1.0 / K)
    a_ref[2 * D_IN:3 * D_IN, :] = jnp.dot(wad_tp[...], m2,
                                          precision=hp) * (1.0 / K)
    c_ref[...] = (bwc[...]
                  + jnp.dot(baa[...], m1, precision=hp)
                  + jnp.dot(bad[...], m2, precision=hp))


def _compose(waa_tp, baa, wad_tp, bad, ws_t, wa_t, wd_t, wc_t, bwc):
    return pl.pallas_call(
        _compose_body,
        out_shape=[
            jax.ShapeDtypeStruct((3 * D_IN, D_OUT), jnp.float32),
            jax.ShapeDtypeStruct((1, D_OUT), jnp.float32),
        ],
    )(waa_tp, baa, wad_tp, bad, ws_t, wa_t, wd_t, wc_t, bwc)


def _tc_body(s_ref, a_sum_ref, d_sum_ref, a_ref, c_ref, o_ref):
    y = (jnp.dot(s_ref[...], a_ref[0:D_IN, :])
         + jnp.dot(a_sum_ref[...], a_ref[D_IN:2 * D_IN, :])
         + jnp.dot(d_sum_ref[...], a_ref[2 * D_IN:3 * D_IN, :])
         + c_ref[...])
    y = jnp.where(y >= 0, y, 0.2 * y)
    nrm = jnp.maximum(jnp.sqrt(jnp.sum(y * y, axis=-1, keepdims=True)), 1e-12)
    o_ref[...] = y / nrm


_TC_BLK = 2048


def _tc_dense(m_self, a_sum, d_sum, a, c):
    def whole(shape):
        return pl.BlockSpec(shape, lambda i: tuple(0 for _ in shape))

    row = lambda w: pl.BlockSpec((_TC_BLK, w), lambda i: (i, 0))
    return pl.pallas_call(
        _tc_body,
        grid=(B // _TC_BLK,),
        in_specs=[
            row(D_IN), row(D_IN), row(D_IN),
            whole((3 * D_IN, D_OUT)), whole((1, D_OUT)),
        ],
        out_specs=pl.BlockSpec((_TC_BLK, D_OUT), lambda i: (i, 0)),
        out_shape=jax.ShapeDtypeStruct((B, D_OUT), jnp.float32),
    )(m_self, a_sum, d_sum, a, c)


def _pack_table(table):
    tb = jax.lax.bitcast_convert_type(table.astype(jnp.bfloat16), jnp.uint16)
    tu = tb.reshape(-1, DW, 2)
    pk = tu[..., 0].astype(jnp.uint32) | (tu[..., 1].astype(jnp.uint32) << 16)
    return jax.lax.bitcast_convert_type(pk, jnp.int32)


def _unpack_rows(pk):
    pku = jax.lax.bitcast_convert_type(pk, jnp.uint32)
    lo = jax.lax.bitcast_convert_type((pku & 0xFFFF).astype(jnp.uint16),
                                      jnp.bfloat16)
    hi = jax.lax.bitcast_convert_type((pku >> 16).astype(jnp.uint16),
                                      jnp.bfloat16)
    return jnp.stack([lo, hi], axis=-1).reshape(-1, D_IN).astype(jnp.float32)


def kernel(nodes, adj_neighbors, dis_neighbors, table,
           W_agg_adj, b_agg_adj, W_agg_dis, b_agg_dis,
           W_self, W_adj, W_dis, WC, b_WC, bias):
    nodes2 = nodes.astype(jnp.int32).reshape(NW * NSC, SELF_CH)
    adj2 = adj_neighbors.astype(jnp.int32).reshape(NW * NCH, CH * K)
    dis2 = dis_neighbors.astype(jnp.int32).reshape(NW * NCH, CH * K)
    tpk = _pack_table(table)
    m_self_pk, a_sum, d_sum = _sc_gather(nodes2, adj2, dis2, tpk)
    m_self = _unpack_rows(m_self_pk)
    a, c = _compose(
        W_agg_adj.T[_PERM], b_agg_adj.reshape(1, D_IN),
        W_agg_dis.T[_PERM], b_agg_dis.reshape(1, D_IN),
        W_self.T, W_adj.T, W_dis.T,
        WC.T, (b_WC + bias).reshape(1, D_OUT),
    )
    return _tc_dense(m_self, a_sum, d_sum, a, c)


# speedup vs baseline: 1.9988x; 1.9988x over previous
import functools

import jax
import jax.numpy as jnp
from jax import lax
from jax.experimental import pallas as pl
from jax.experimental.pallas import tpu as pltpu
from jax.experimental.pallas import tpu_sc as plsc

N_NODES = 100000
D_IN = 128
D_OUT = 384
D3 = D_OUT // 3
B = 16384
K = 16
DW = D_IN // 2

NC = 2
NS = 16
NW = NC * NS
RPW = B // NW
CH = 8
NCH = RPW // CH
NT = 2 * NCH
SELF_CH = 128
NSC = RPW // SELF_CH
VT = NT + NSC
NBUF = 6

_MASK_HI = -65536


def _sc_body(nodes2_hbm, adj2_hbm, dis2_hbm, tpk_hbm,
             selfpk_out, adj_out, dis_out,
             idx_adj, idx_dis, idx_self,
             rows0, rows1, rows2, rows3, rows4, rows5, ob0, ob1,
             semg0, semg1, semg2, semg3, semg4, semg5,
             semw0, semw1, semself):
    wid = lax.axis_index("s") * NC + lax.axis_index("c")
    base = wid * RPW

    rows = (rows0, rows1, rows2, rows3, rows4, rows5)
    semg = (semg0, semg1, semg2, semg3, semg4, semg5)
    ob = (ob0, ob1)
    semw = (semw0, semw1)

    pltpu.sync_copy(adj2_hbm.at[pl.ds(wid * NCH, NCH), :], idx_adj)
    pltpu.sync_copy(dis2_hbm.at[pl.ds(wid * NCH, NCH), :], idx_dis)
    pltpu.sync_copy(nodes2_hbm.at[pl.ds(wid * NSC, NSC), :], idx_self)

    def fire(t, slot):
        @pl.when(t < NCH)
        def _():
            pltpu.async_copy(tpk_hbm.at[idx_adj.at[t]], rows[slot], semg[slot])

        @pl.when(jnp.logical_and(t >= NCH, t < NT))
        def _():
            pltpu.async_copy(tpk_hbm.at[idx_dis.at[t - NCH]], rows[slot],
                             semg[slot])

        @pl.when(jnp.logical_and(t >= NT, t < VT))
        def _():
            pltpu.async_copy(tpk_hbm.at[idx_self.at[t - NT]], rows[slot],
                             semg[slot])

    def reduce_chunk(slot, oslot):
        def red_node(j, carry):
            rb = j * K
            for g in range(DW // 16):
                col = g * 16
                w = rows[slot][rb, pl.ds(col, 16)]
                acc_e = plsc.bitcast(w << 16, jnp.float32)
                acc_o = plsc.bitcast(w & _MASK_HI, jnp.float32)
                for i in range(1, K):
                    w = rows[slot][rb + i, pl.ds(col, 16)]
                    acc_e = acc_e + plsc.bitcast(w << 16, jnp.float32)
                    acc_o = acc_o + plsc.bitcast(w & _MASK_HI, jnp.float32)
                ob[oslot][j, pl.ds(col, 16)] = acc_e
                ob[oslot][j, pl.ds(DW + col, 16)] = acc_o
            return carry

        lax.fori_loop(0, CH, red_node, 0)

    for s in range(NBUF - 1):
        fire(s, s)

    @pl.loop(0, VT, step=NBUF)
    def outer(t0):
        for b in range(NBUF):
            t = t0 + b
            fire(t + NBUF - 1, (b + NBUF - 1) % NBUF)
            pltpu.make_async_copy(tpk_hbm.at[idx_adj.at[0]], rows[b],
                                  semg[b]).wait()

            @pl.when(t < NT)
            def _():
                @pl.when(t >= 2)
                def _():
                    pltpu.make_async_copy(ob[b % 2], adj_out.at[pl.ds(0, CH)],
                                          semw[b % 2]).wait()

                reduce_chunk(b, b % 2)

                @pl.when(t < NCH)
                def _():
                    pltpu.async_copy(ob[b % 2],
                                     adj_out.at[pl.ds(base + t * CH, CH)],
                                     semw[b % 2])

                @pl.when(t >= NCH)
                def _():
                    pltpu.async_copy(
                        ob[b % 2],
                        dis_out.at[pl.ds(base + (t - NCH) * CH, CH)],
                        semw[b % 2])

            @pl.when(jnp.logical_and(t >= NT, t < VT))
            def _():
                off = base + (t - NT) * SELF_CH
                pltpu.async_copy(rows[b], selfpk_out.at[pl.ds(off, SELF_CH)],
                                 semself)

    for i in range(2):
        t = NT - 2 + i
        pltpu.make_async_copy(ob[t % 2],
                              dis_out.at[pl.ds(base + (t - NCH) * CH, CH)],
                              semw[t % 2]).wait()
    for c in range(NSC):
        pltpu.make_async_copy(
            rows[0], selfpk_out.at[pl.ds(base + c * SELF_CH, SELF_CH)],
            semself).wait()


_sc_gather = functools.partial(
    pl.kernel,
    out_type=[
        jax.ShapeDtypeStruct((B, DW), jnp.int32),
        jax.ShapeDtypeStruct((B, D_IN), jnp.float32),
        jax.ShapeDtypeStruct((B, D_IN), jnp.float32),
    ],
    mesh=plsc.VectorSubcoreMesh(core_axis_name="c", subcore_axis_name="s"),
    compiler_params=pltpu.CompilerParams(needs_layout_passes=False,
                                         use_tc_tiling_on_sc=False),
    scratch_types=(
        [pltpu.VMEM((NCH, CH * K), jnp.int32)] * 2
        + [pltpu.VMEM((NSC, SELF_CH), jnp.int32)]
        + [pltpu.VMEM((CH * K, DW), jnp.int32)] * NBUF
        + [pltpu.VMEM((CH, D_IN), jnp.float32)] * 2
        + [pltpu.SemaphoreType.DMA] * (NBUF + 3)
    ),
)(_sc_body)


def _compose_body(waa_tp, baa, wad_tp, bad, ws_t, wa_t, wd_t, wc_t, bwc,
                  a_ref, c_ref):
    hp = jax.lax.Precision.HIGHEST
    m1 = jnp.dot(wa_t[...], wc_t[D3:2 * D3, :], precision=hp)
    m2 = jnp.dot(wd_t[...], wc_t[2 * D3:D_OUT, :], precision=hp)
    a_ref[0:D_IN, :] = jnp.dot(ws_t[...], wc_t[0:D3, :], precision=hp)
    a_ref[D_IN:2 * D_IN, :] = jnp.dot(waa_tp[...], m1,
                                      precision=hp) * (1.0 / K)
    a_ref[2 * D_IN:3 * D_IN, :] = jnp.dot(wad_tp[...], m2,
                                          precision=hp) * (1.0 / K)
    c_ref[...] = (bwc[...]
                  + jnp.dot(baa[...], m1, precision=hp)
                  + jnp.dot(bad[...], m2, precision=hp))


def _compose(waa_tp, baa, wad_tp, bad, ws_t, wa_t, wd_t, wc_t, bwc):
    return pl.pallas_call(
        _compose_body,
        out_shape=[
            jax.ShapeDtypeStruct((3 * D_IN, D_OUT), jnp.float32),
            jax.ShapeDtypeStruct((1, D_OUT), jnp.float32),
        ],
    )(waa_tp, baa, wad_tp, bad, ws_t, wa_t, wd_t, wc_t, bwc)


def _tc_body(s_ref, a_sum_ref, d_sum_ref, a_ref, c_ref, o_ref):
    y = (jnp.dot(s_ref[...], a_ref[0:D_IN, :])
         + jnp.dot(a_sum_ref[...], a_ref[D_IN:2 * D_IN, :])
         + jnp.dot(d_sum_ref[...], a_ref[2 * D_IN:3 * D_IN, :])
         + c_ref[...])
    y = jnp.where(y >= 0, y, 0.2 * y)
    nrm = jnp.maximum(jnp.sqrt(jnp.sum(y * y, axis=-1, keepdims=True)), 1e-12)
    o_ref[...] = y / nrm


_TC_BLK = 2048


def _tc_dense(m_self, a_sum, d_sum, a, c):
    def whole(shape):
        return pl.BlockSpec(shape, lambda i: tuple(0 for _ in shape))

    row = lambda w: pl.BlockSpec((_TC_BLK, w), lambda i: (i, 0))
    return pl.pallas_call(
        _tc_body,
        grid=(B // _TC_BLK,),
        in_specs=[
            row(D_IN), row(D_IN), row(D_IN),
            whole((3 * D_IN, D_OUT)), whole((1, D_OUT)),
        ],
        out_specs=pl.BlockSpec((_TC_BLK, D_OUT), lambda i: (i, 0)),
        out_shape=jax.ShapeDtypeStruct((B, D_OUT), jnp.float32),
    )(m_self, a_sum, d_sum, a, c)


def _pack_table(table):
    tb = jax.lax.bitcast_convert_type(table.astype(jnp.bfloat16), jnp.uint16)
    pk = (tb[:, :DW].astype(jnp.uint32)
          | (tb[:, DW:].astype(jnp.uint32) << 16))
    return jax.lax.bitcast_convert_type(pk, jnp.int32)


def _unpack_rows(pk):
    pku = jax.lax.bitcast_convert_type(pk, jnp.uint32)
    lo = jax.lax.bitcast_convert_type((pku & 0xFFFF).astype(jnp.uint16),
                                      jnp.bfloat16)
    hi = jax.lax.bitcast_convert_type((pku >> 16).astype(jnp.uint16),
                                      jnp.bfloat16)
    return jnp.concatenate([lo, hi], axis=-1).astype(jnp.float32)


def kernel(nodes, adj_neighbors, dis_neighbors, table,
           W_agg_adj, b_agg_adj, W_agg_dis, b_agg_dis,
           W_self, W_adj, W_dis, WC, b_WC, bias):
    nodes2 = nodes.astype(jnp.int32).reshape(NW * NSC, SELF_CH)
    adj2 = adj_neighbors.astype(jnp.int32).reshape(NW * NCH, CH * K)
    dis2 = dis_neighbors.astype(jnp.int32).reshape(NW * NCH, CH * K)
    tpk = _pack_table(table)
    m_self_pk, a_sum, d_sum = _sc_gather(nodes2, adj2, dis2, tpk)
    m_self = _unpack_rows(m_self_pk)
    a, c = _compose(
        W_agg_adj.T, b_agg_adj.reshape(1, D_IN),
        W_agg_dis.T, b_agg_dis.reshape(1, D_IN),
        W_self.T, W_adj.T, W_dis.T,
        WC.T, (b_WC + bias).reshape(1, D_OUT),
    )
    return _tc_dense(m_self, a_sum, d_sum, a, c)


# speedup vs baseline: 2.1398x; 1.0705x over previous
import functools

import jax
import jax.numpy as jnp
from jax import lax
from jax.experimental import pallas as pl
from jax.experimental.pallas import tpu as pltpu
from jax.experimental.pallas import tpu_sc as plsc

N_NODES = 100000
D_IN = 128
D_OUT = 384
D3 = D_OUT // 3
B = 16384
K = 16
DW = D_IN // 2

NC = 2
NS = 16
NW = NC * NS
RPW = B // NW
CH = 16
NCH = RPW // CH
NT = 2 * NCH
SELF_CH = 256
NSC = RPW // SELF_CH
VT = NT + NSC
NBUF = 6

_MASK_HI = -65536


def _sc_body(nodes2_hbm, adj2_hbm, dis2_hbm, tpk_hbm,
             selfpk_out, adj_out, dis_out,
             idx_adj, idx_dis, idx_self,
             rows0, rows1, rows2, rows3, rows4, rows5, ob0, ob1,
             semg0, semg1, semg2, semg3, semg4, semg5,
             semw0, semw1, semself):
    wid = lax.axis_index("s") * NC + lax.axis_index("c")
    base = wid * RPW

    rows = (rows0, rows1, rows2, rows3, rows4, rows5)
    semg = (semg0, semg1, semg2, semg3, semg4, semg5)
    ob = (ob0, ob1)
    semw = (semw0, semw1)

    pltpu.sync_copy(adj2_hbm.at[pl.ds(wid * NCH, NCH), :], idx_adj)
    pltpu.sync_copy(dis2_hbm.at[pl.ds(wid * NCH, NCH), :], idx_dis)
    pltpu.sync_copy(nodes2_hbm.at[pl.ds(wid * NSC, NSC), :], idx_self)

    def fire(t, slot):
        @pl.when(t < NCH)
        def _():
            pltpu.async_copy(tpk_hbm.at[idx_adj.at[t]], rows[slot], semg[slot])

        @pl.when(jnp.logical_and(t >= NCH, t < NT))
        def _():
            pltpu.async_copy(tpk_hbm.at[idx_dis.at[t - NCH]], rows[slot],
                             semg[slot])

        @pl.when(jnp.logical_and(t >= NT, t < VT))
        def _():
            pltpu.async_copy(tpk_hbm.at[idx_self.at[t - NT]], rows[slot],
                             semg[slot])

    def reduce_chunk(slot, oslot):
        def red_node(j, carry):
            rb = j * K
            for g in range(DW // 16):
                col = g * 16
                w = rows[slot][rb, pl.ds(col, 16)]
                acc_e = plsc.bitcast(w << 16, jnp.float32)
                acc_o = plsc.bitcast(w & _MASK_HI, jnp.float32)
                for i in range(1, K):
                    w = rows[slot][rb + i, pl.ds(col, 16)]
                    acc_e = acc_e + plsc.bitcast(w << 16, jnp.float32)
                    acc_o = acc_o + plsc.bitcast(w & _MASK_HI, jnp.float32)
                ob[oslot][j, pl.ds(col, 16)] = acc_e
                ob[oslot][j, pl.ds(DW + col, 16)] = acc_o
            return carry

        lax.fori_loop(0, CH, red_node, 0)

    for s in range(NBUF - 1):
        fire(s, s)

    @pl.loop(0, VT, step=NBUF)
    def outer(t0):
        for b in range(NBUF):
            t = t0 + b
            fire(t + NBUF - 1, (b + NBUF - 1) % NBUF)
            pltpu.make_async_copy(tpk_hbm.at[idx_adj.at[0]], rows[b],
                                  semg[b]).wait()

            @pl.when(t < NT)
            def _():
                @pl.when(t >= 2)
                def _():
                    pltpu.make_async_copy(ob[b % 2], adj_out.at[pl.ds(0, CH)],
                                          semw[b % 2]).wait()

                reduce_chunk(b, b % 2)

                @pl.when(t < NCH)
                def _():
                    pltpu.async_copy(ob[b % 2],
                                     adj_out.at[pl.ds(base + t * CH, CH)],
                                     semw[b % 2])

                @pl.when(t >= NCH)
                def _():
                    pltpu.async_copy(
                        ob[b % 2],
                        dis_out.at[pl.ds(base + (t - NCH) * CH, CH)],
                        semw[b % 2])

            @pl.when(jnp.logical_and(t >= NT, t < VT))
            def _():
                off = base + (t - NT) * SELF_CH
                pltpu.async_copy(rows[b], selfpk_out.at[pl.ds(off, SELF_CH)],
                                 semself)

    for i in range(2):
        t = NT - 2 + i
        pltpu.make_async_copy(ob[t % 2],
                              dis_out.at[pl.ds(base + (t - NCH) * CH, CH)],
                              semw[t % 2]).wait()
    for c in range(NSC):
        pltpu.make_async_copy(
            rows[0], selfpk_out.at[pl.ds(base + c * SELF_CH, SELF_CH)],
            semself).wait()


_sc_gather = functools.partial(
    pl.kernel,
    out_type=[
        jax.ShapeDtypeStruct((B, DW), jnp.int32),
        jax.ShapeDtypeStruct((B, D_IN), jnp.float32),
        jax.ShapeDtypeStruct((B, D_IN), jnp.float32),
    ],
    mesh=plsc.VectorSubcoreMesh(core_axis_name="c", subcore_axis_name="s"),
    compiler_params=pltpu.CompilerParams(needs_layout_passes=False,
                                         use_tc_tiling_on_sc=False),
    scratch_types=(
        [pltpu.VMEM((NCH, CH * K), jnp.int32)] * 2
        + [pltpu.VMEM((NSC, SELF_CH), jnp.int32)]
        + [pltpu.VMEM((CH * K, DW), jnp.int32)] * NBUF
        + [pltpu.VMEM((CH, D_IN), jnp.float32)] * 2
        + [pltpu.SemaphoreType.DMA] * (NBUF + 3)
    ),
)(_sc_body)


def _compose_body(waa_tp, baa, wad_tp, bad, ws_t, wa_t, wd_t, wc_t, bwc,
                  a_ref, c_ref):
    hp = jax.lax.Precision.HIGHEST
    m1 = jnp.dot(wa_t[...], wc_t[D3:2 * D3, :], precision=hp)
    m2 = jnp.dot(wd_t[...], wc_t[2 * D3:D_OUT, :], precision=hp)
    a_ref[0:D_IN, :] = jnp.dot(ws_t[...], wc_t[0:D3, :], precision=hp)
    a_ref[D_IN:2 * D_IN, :] = jnp.dot(waa_tp[...], m1,
                                      precision=hp) * (1.0 / K)
    a_ref[2 * D_IN:3 * D_IN, :] = jnp.dot(wad_tp[...], m2,
                                          precision=hp) * (1.0 / K)
    c_ref[...] = (bwc[...]
                  + jnp.dot(baa[...], m1, precision=hp)
                  + jnp.dot(bad[...], m2, precision=hp))


def _compose(waa_tp, baa, wad_tp, bad, ws_t, wa_t, wd_t, wc_t, bwc):
    return pl.pallas_call(
        _compose_body,
        out_shape=[
            jax.ShapeDtypeStruct((3 * D_IN, D_OUT), jnp.float32),
            jax.ShapeDtypeStruct((1, D_OUT), jnp.float32),
        ],
    )(waa_tp, baa, wad_tp, bad, ws_t, wa_t, wd_t, wc_t, bwc)


def _tc_body(s_ref, a_sum_ref, d_sum_ref, a_ref, c_ref, o_ref):
    y = (jnp.dot(s_ref[...], a_ref[0:D_IN, :])
         + jnp.dot(a_sum_ref[...], a_ref[D_IN:2 * D_IN, :])
         + jnp.dot(d_sum_ref[...], a_ref[2 * D_IN:3 * D_IN, :])
         + c_ref[...])
    y = jnp.where(y >= 0, y, 0.2 * y)
    nrm = jnp.maximum(jnp.sqrt(jnp.sum(y * y, axis=-1, keepdims=True)), 1e-12)
    o_ref[...] = y / nrm


_TC_BLK = 2048


def _tc_dense(m_self, a_sum, d_sum, a, c):
    def whole(shape):
        return pl.BlockSpec(shape, lambda i: tuple(0 for _ in shape))

    row = lambda w: pl.BlockSpec((_TC_BLK, w), lambda i: (i, 0))
    return pl.pallas_call(
        _tc_body,
        grid=(B // _TC_BLK,),
        in_specs=[
            row(D_IN), row(D_IN), row(D_IN),
            whole((3 * D_IN, D_OUT)), whole((1, D_OUT)),
        ],
        out_specs=pl.BlockSpec((_TC_BLK, D_OUT), lambda i: (i, 0)),
        out_shape=jax.ShapeDtypeStruct((B, D_OUT), jnp.float32),
    )(m_self, a_sum, d_sum, a, c)


def _pack_table(table):
    tb = jax.lax.bitcast_convert_type(table.astype(jnp.bfloat16), jnp.uint16)
    pk = (tb[:, :DW].astype(jnp.uint32)
          | (tb[:, DW:].astype(jnp.uint32) << 16))
    return jax.lax.bitcast_convert_type(pk, jnp.int32)


def _unpack_rows(pk):
    pku = jax.lax.bitcast_convert_type(pk, jnp.uint32)
    lo = jax.lax.bitcast_convert_type((pku & 0xFFFF).astype(jnp.uint16),
                                      jnp.bfloat16)
    hi = jax.lax.bitcast_convert_type((pku >> 16).astype(jnp.uint16),
                                      jnp.bfloat16)
    return jnp.concatenate([lo, hi], axis=-1).astype(jnp.float32)


def kernel(nodes, adj_neighbors, dis_neighbors, table,
           W_agg_adj, b_agg_adj, W_agg_dis, b_agg_dis,
           W_self, W_adj, W_dis, WC, b_WC, bias):
    nodes2 = nodes.astype(jnp.int32).reshape(NW * NSC, SELF_CH)
    adj2 = adj_neighbors.astype(jnp.int32).reshape(NW * NCH, CH * K)
    dis2 = dis_neighbors.astype(jnp.int32).reshape(NW * NCH, CH * K)
    tpk = _pack_table(table)
    m_self_pk, a_sum, d_sum = _sc_gather(nodes2, adj2, dis2, tpk)
    m_self = _unpack_rows(m_self_pk)
    a, c = _compose(
        W_agg_adj.T, b_agg_adj.reshape(1, D_IN),
        W_agg_dis.T, b_agg_dis.reshape(1, D_IN),
        W_self.T, W_adj.T, W_dis.T,
        WC.T, (b_WC + bias).reshape(1, D_OUT),
    )
    return _tc_dense(m_self, a_sum, d_sum, a, c)


# speedup vs baseline: 2.2082x; 1.0320x over previous
import functools

import jax
import jax.numpy as jnp
from jax import lax
from jax.experimental import pallas as pl
from jax.experimental.pallas import tpu as pltpu
from jax.experimental.pallas import tpu_sc as plsc

N_NODES = 100000
D_IN = 128
D_OUT = 384
D3 = D_OUT // 3
B = 16384
K = 16
DW = D_IN // 2

NC = 2
NS = 16
NW = NC * NS
RPW = B // NW
CH = 16
NCH = RPW // CH
NT = 2 * NCH
SELF_CH = 256
NSC = RPW // SELF_CH
VT = NT + NSC
NBUF = 4
LOOP_HI = ((VT + NBUF - 1) // NBUF) * NBUF

_MASK_HI = -65536


def _sc_body(nodes2_hbm, adj2_hbm, dis2_hbm, tpk_hbm,
             self_out, adj_out, dis_out,
             idx_adj, idx_dis, idx_self,
             rows0, rows1, rows2, rows3, selfbuf, ob0, ob1,
             semg0, semg1, semg2, semg3, semw0, semw1, semself):
    wid = lax.axis_index("s") * NC + lax.axis_index("c")
    base = wid * RPW

    rows = (rows0, rows1, rows2, rows3)
    semg = (semg0, semg1, semg2, semg3)
    ob = (ob0, ob1)
    semw = (semw0, semw1)

    pltpu.sync_copy(adj2_hbm.at[pl.ds(wid * NCH, NCH), :], idx_adj)
    pltpu.sync_copy(dis2_hbm.at[pl.ds(wid * NCH, NCH), :], idx_dis)
    pltpu.sync_copy(nodes2_hbm.at[pl.ds(wid * NSC, NSC), :], idx_self)

    def fire(t, slot):
        @pl.when(t < NCH)
        def _():
            pltpu.async_copy(tpk_hbm.at[idx_adj.at[t]], rows[slot], semg[slot])

        @pl.when(jnp.logical_and(t >= NCH, t < NT))
        def _():
            pltpu.async_copy(tpk_hbm.at[idx_dis.at[t - NCH]], rows[slot],
                             semg[slot])

        @pl.when(jnp.logical_and(t >= NT, t < VT))
        def _():
            pltpu.async_copy(tpk_hbm.at[idx_self.at[t - NT]], rows[slot],
                             semg[slot])

    def widen(w):
        return (plsc.bitcast(w << 16, jnp.float32),
                plsc.bitcast(w & _MASK_HI, jnp.float32))

    def reduce_chunk(slot, oslot):
        def red_node(j, carry):
            rb = j * K
            for g in range(DW // 16):
                col = g * 16
                acc_e, acc_o = widen(rows[slot][rb, pl.ds(col, 16)])
                for i in range(1, K):
                    lo, hi = widen(rows[slot][rb + i, pl.ds(col, 16)])
                    acc_e = acc_e + lo
                    acc_o = acc_o + hi
                ob[oslot][j, pl.ds(col, 16)] = acc_e
                ob[oslot][j, pl.ds(DW + col, 16)] = acc_o
            return carry

        lax.fori_loop(0, CH, red_node, 0)

    def widen_chunk(slot):
        def wide_row(j, carry):
            for g in range(DW // 16):
                col = g * 16
                lo, hi = widen(rows[slot][j, pl.ds(col, 16)])
                selfbuf[j, pl.ds(col, 16)] = lo
                selfbuf[j, pl.ds(DW + col, 16)] = hi
            return carry

        lax.fori_loop(0, SELF_CH, wide_row, 0)

    for s in range(NBUF - 1):
        fire(s, s)

    @pl.loop(0, LOOP_HI, step=NBUF)
    def outer(t0):
        for b in range(NBUF):
            t = t0 + b
            fire(t + NBUF - 1, (b + NBUF - 1) % NBUF)

            @pl.when(t < VT)
            def _():
                pltpu.make_async_copy(tpk_hbm.at[idx_adj.at[0]], rows[b],
                                      semg[b]).wait()

                @pl.when(t < NT)
                def _():
                    @pl.when(t >= 2)
                    def _():
                        pltpu.make_async_copy(ob[b % 2],
                                              adj_out.at[pl.ds(0, CH)],
                                              semw[b % 2]).wait()

                    reduce_chunk(b, b % 2)

                    @pl.when(t < NCH)
                    def _():
                        pltpu.async_copy(ob[b % 2],
                                         adj_out.at[pl.ds(base + t * CH, CH)],
                                         semw[b % 2])

                    @pl.when(t >= NCH)
                    def _():
                        pltpu.async_copy(
                            ob[b % 2],
                            dis_out.at[pl.ds(base + (t - NCH) * CH, CH)],
                            semw[b % 2])

                @pl.when(t >= NT)
                def _():
                    @pl.when(t > NT)
                    def _():
                        pltpu.make_async_copy(
                            selfbuf, self_out.at[pl.ds(0, SELF_CH)],
                            semself).wait()

                    widen_chunk(b)
                    off = base + (t - NT) * SELF_CH
                    pltpu.async_copy(selfbuf,
                                     self_out.at[pl.ds(off, SELF_CH)],
                                     semself)

    for i in range(2):
        t = NT - 2 + i
        pltpu.make_async_copy(ob[t % 2],
                              dis_out.at[pl.ds(base + (t - NCH) * CH, CH)],
                              semw[t % 2]).wait()
    pltpu.make_async_copy(
        selfbuf, self_out.at[pl.ds(base + (NSC - 1) * SELF_CH, SELF_CH)],
        semself).wait()


_sc_gather = functools.partial(
    pl.kernel,
    out_type=[
        jax.ShapeDtypeStruct((B, D_IN), jnp.float32),
        jax.ShapeDtypeStruct((B, D_IN), jnp.float32),
        jax.ShapeDtypeStruct((B, D_IN), jnp.float32),
    ],
    mesh=plsc.VectorSubcoreMesh(core_axis_name="c", subcore_axis_name="s"),
    compiler_params=pltpu.CompilerParams(needs_layout_passes=False,
                                         use_tc_tiling_on_sc=False),
    scratch_types=(
        [pltpu.VMEM((NCH, CH * K), jnp.int32)] * 2
        + [pltpu.VMEM((NSC, SELF_CH), jnp.int32)]
        + [pltpu.VMEM((CH * K, DW), jnp.int32)] * NBUF
        + [pltpu.VMEM((SELF_CH, D_IN), jnp.float32)]
        + [pltpu.VMEM((CH, D_IN), jnp.float32)] * 2
        + [pltpu.SemaphoreType.DMA] * (NBUF + 3)
    ),
)(_sc_body)


def _pack_body(x_ref, o_ref):
    xb = jax.lax.bitcast_convert_type(x_ref[...], jnp.int32)
    lo = xb[:, :DW]
    hi = xb[:, DW:]
    lo_r = (lo + 0x7FFF + ((lo >> 16) & 1)) >> 16
    hi_r = (hi + 0x7FFF + ((hi >> 16) & 1)) & _MASK_HI
    o_ref[...] = (lo_r & 0xFFFF) | hi_r


_PACK_BLK = 2000


def _pack_table(table):
    return pl.pallas_call(
        _pack_body,
        grid=(N_NODES // _PACK_BLK,),
        in_specs=[pl.BlockSpec((_PACK_BLK, D_IN), lambda i: (i, 0))],
        out_specs=pl.BlockSpec((_PACK_BLK, DW), lambda i: (i, 0)),
        out_shape=jax.ShapeDtypeStruct((N_NODES, DW), jnp.int32),
    )(table)


def _compose_body(waa_t, baa, wad_t, bad, ws_t, wa_t, wd_t, wc_t, bwc,
                  a_ref, c_ref):
    hp = jax.lax.Precision.HIGHEST
    m1 = jnp.dot(wa_t[...], wc_t[D3:2 * D3, :], precision=hp)
    m2 = jnp.dot(wd_t[...], wc_t[2 * D3:D_OUT, :], precision=hp)
    a_ref[0:D_IN, :] = jnp.dot(ws_t[...], wc_t[0:D3, :], precision=hp)
    a_ref[D_IN:2 * D_IN, :] = jnp.dot(waa_t[...], m1, precision=hp) * (1.0 / K)
    a_ref[2 * D_IN:3 * D_IN, :] = jnp.dot(wad_t[...], m2,
                                          precision=hp) * (1.0 / K)
    c_ref[...] = (bwc[...]
                  + jnp.dot(baa[...], m1, precision=hp)
                  + jnp.dot(bad[...], m2, precision=hp))


def _compose(waa_t, baa, wad_t, bad, ws_t, wa_t, wd_t, wc_t, bwc):
    return pl.pallas_call(
        _compose_body,
        out_shape=[
            jax.ShapeDtypeStruct((3 * D_IN, D_OUT), jnp.float32),
            jax.ShapeDtypeStruct((1, D_OUT), jnp.float32),
        ],
    )(waa_t, baa, wad_t, bad, ws_t, wa_t, wd_t, wc_t, bwc)


def _tc_body(s_ref, a_sum_ref, d_sum_ref, a_ref, c_ref, o_ref):
    y = (jnp.dot(s_ref[...], a_ref[0:D_IN, :])
         + jnp.dot(a_sum_ref[...], a_ref[D_IN:2 * D_IN, :])
         + jnp.dot(d_sum_ref[...], a_ref[2 * D_IN:3 * D_IN, :])
         + c_ref[...])
    y = jnp.where(y >= 0, y, 0.2 * y)
    nrm = jnp.maximum(jnp.sqrt(jnp.sum(y * y, axis=-1, keepdims=True)), 1e-12)
    o_ref[...] = y / nrm


_TC_BLK = 2048


def _tc_dense(m_self, a_sum, d_sum, a, c):
    def whole(shape):
        return pl.BlockSpec(shape, lambda i: tuple(0 for _ in shape))

    row = lambda w: pl.BlockSpec((_TC_BLK, w), lambda i: (i, 0))
    return pl.pallas_call(
        _tc_body,
        grid=(B // _TC_BLK,),
        in_specs=[
            row(D_IN), row(D_IN), row(D_IN),
            whole((3 * D_IN, D_OUT)), whole((1, D_OUT)),
        ],
        out_specs=pl.BlockSpec((_TC_BLK, D_OUT), lambda i: (i, 0)),
        out_shape=jax.ShapeDtypeStruct((B, D_OUT), jnp.float32),
    )(m_self, a_sum, d_sum, a, c)


def kernel(nodes, adj_neighbors, dis_neighbors, table,
           W_agg_adj, b_agg_adj, W_agg_dis, b_agg_dis,
           W_self, W_adj, W_dis, WC, b_WC, bias):
    nodes2 = nodes.astype(jnp.int32).reshape(NW * NSC, SELF_CH)
    adj2 = adj_neighbors.astype(jnp.int32).reshape(NW * NCH, CH * K)
    dis2 = dis_neighbors.astype(jnp.int32).reshape(NW * NCH, CH * K)
    tpk = _pack_table(table)
    m_self, a_sum, d_sum = _sc_gather(nodes2, adj2, dis2, tpk)
    a, c = _compose(
        W_agg_adj.T, b_agg_adj.reshape(1, D_IN),
        W_agg_dis.T, b_agg_dis.reshape(1, D_IN),
        W_self.T, W_adj.T, W_dis.T,
        WC.T, (b_WC + bias).reshape(1, D_OUT),
    )
    return _tc_dense(m_self, a_sum, d_sum, a, c)


# speedup vs baseline: 2.3258x; 1.0533x over previous
import functools

import jax
import jax.numpy as jnp
from jax import lax
from jax.experimental import pallas as pl
from jax.experimental.pallas import tpu as pltpu
from jax.experimental.pallas import tpu_sc as plsc

N_NODES = 100000
D_IN = 128
D_OUT = 384
D3 = D_OUT // 3
B = 16384
K = 16

NC = 2
NS = 16
NW = NC * NS
RPW = B // NW
CH = 16
NCH = RPW // CH
NT = 2 * NCH
SELF_CH = 256
NSC = RPW // SELF_CH
VT = NT + NSC
NBUF = 3


def _sc_body(nodes2_hbm, adj2_hbm, dis2_hbm, table_hbm,
             self_out, adj_out, dis_out,
             idx_adj, idx_dis, idx_self,
             rows0, rows1, rows2, ob0, ob1,
             semg0, semg1, semg2, semw0, semw1, semself):
    wid = lax.axis_index("s") * NC + lax.axis_index("c")
    base = wid * RPW

    rows = (rows0, rows1, rows2)
    semg = (semg0, semg1, semg2)
    ob = (ob0, ob1)
    semw = (semw0, semw1)

    pltpu.sync_copy(adj2_hbm.at[pl.ds(wid * NCH, NCH), :], idx_adj)
    pltpu.sync_copy(dis2_hbm.at[pl.ds(wid * NCH, NCH), :], idx_dis)
    pltpu.sync_copy(nodes2_hbm.at[pl.ds(wid * NSC, NSC), :], idx_self)

    def fire(t, slot):
        @pl.when(t < NCH)
        def _():
            pltpu.async_copy(table_hbm.at[idx_adj.at[t]], rows[slot],
                             semg[slot])

        @pl.when(jnp.logical_and(t >= NCH, t < NT))
        def _():
            pltpu.async_copy(table_hbm.at[idx_dis.at[t - NCH]], rows[slot],
                             semg[slot])

        @pl.when(jnp.logical_and(t >= NT, t < VT))
        def _():
            pltpu.async_copy(table_hbm.at[idx_self.at[t - NT]], rows[slot],
                             semg[slot])

    def reduce_chunk(slot, oslot):
        def red_node(j, carry):
            rb = j * K
            for g in range(D_IN // 16):
                col = g * 16
                acc = rows[slot][rb, pl.ds(col, 16)]
                for i in range(1, K):
                    acc = acc + rows[slot][rb + i, pl.ds(col, 16)]
                ob[oslot][j, pl.ds(col, 16)] = acc
            return carry

        lax.fori_loop(0, CH, red_node, 0)

    for s in range(NBUF - 1):
        fire(s, s)

    @pl.loop(0, VT, step=NBUF)
    def outer(t0):
        for b in range(NBUF):
            t = t0 + b
            fire(t + NBUF - 1, (b + NBUF - 1) % NBUF)
            pltpu.make_async_copy(table_hbm.at[idx_adj.at[0]], rows[b],
                                  semg[b]).wait()

            @pl.when(t < NT)
            def _():
                @pl.when(t >= 2)
                def _():
                    pltpu.make_async_copy(ob[b % 2], adj_out.at[pl.ds(0, CH)],
                                          semw[b % 2]).wait()

                reduce_chunk(b, b % 2)

                @pl.when(t < NCH)
                def _():
                    pltpu.async_copy(ob[b % 2],
                                     adj_out.at[pl.ds(base + t * CH, CH)],
                                     semw[b % 2])

                @pl.when(t >= NCH)
                def _():
                    pltpu.async_copy(
                        ob[b % 2],
                        dis_out.at[pl.ds(base + (t - NCH) * CH, CH)],
                        semw[b % 2])

            @pl.when(jnp.logical_and(t >= NT, t < VT))
            def _():
                off = base + (t - NT) * SELF_CH
                pltpu.async_copy(rows[b], self_out.at[pl.ds(off, SELF_CH)],
                                 semself)

    for i in range(2):
        t = NT - 2 + i
        pltpu.make_async_copy(ob[t % 2],
                              dis_out.at[pl.ds(base + (t - NCH) * CH, CH)],
                              semw[t % 2]).wait()
    for c in range(NSC):
        pltpu.make_async_copy(
            rows[0], self_out.at[pl.ds(base + c * SELF_CH, SELF_CH)],
            semself).wait()


_sc_gather = functools.partial(
    pl.kernel,
    out_type=[
        jax.ShapeDtypeStruct((B, D_IN), jnp.float32),
        jax.ShapeDtypeStruct((B, D_IN), jnp.float32),
        jax.ShapeDtypeStruct((B, D_IN), jnp.float32),
    ],
    mesh=plsc.VectorSubcoreMesh(core_axis_name="c", subcore_axis_name="s"),
    compiler_params=pltpu.CompilerParams(use_tc_tiling_on_sc=False),
    scratch_types=(
        [pltpu.VMEM((NCH, CH * K), jnp.int32)] * 2
        + [pltpu.VMEM((NSC, SELF_CH), jnp.int32)]
        + [pltpu.VMEM((CH * K, D_IN), jnp.float32)] * NBUF
        + [pltpu.VMEM((CH, D_IN), jnp.float32)] * 2
        + [pltpu.SemaphoreType.DMA] * (NBUF + 3)
    ),
)(_sc_body)


def _compose_body(waa_t, baa, wad_t, bad, ws_t, wa_t, wd_t, wc_t, bwc,
                  a_ref, c_ref):
    hp = jax.lax.Precision.HIGHEST
    m1 = jnp.dot(wa_t[...], wc_t[D3:2 * D3, :], precision=hp)
    m2 = jnp.dot(wd_t[...], wc_t[2 * D3:D_OUT, :], precision=hp)
    a_ref[0:D_IN, :] = jnp.dot(ws_t[...], wc_t[0:D3, :], precision=hp)
    a_ref[D_IN:2 * D_IN, :] = jnp.dot(waa_t[...], m1, precision=hp) * (1.0 / K)
    a_ref[2 * D_IN:3 * D_IN, :] = jnp.dot(wad_t[...], m2,
                                          precision=hp) * (1.0 / K)
    c_ref[...] = (bwc[...]
                  + jnp.dot(baa[...], m1, precision=hp)
                  + jnp.dot(bad[...], m2, precision=hp))


def _compose(waa_t, baa, wad_t, bad, ws_t, wa_t, wd_t, wc_t, bwc):
    return pl.pallas_call(
        _compose_body,
        out_shape=[
            jax.ShapeDtypeStruct((3 * D_IN, D_OUT), jnp.float32),
            jax.ShapeDtypeStruct((1, D_OUT), jnp.float32),
        ],
    )(waa_t, baa, wad_t, bad, ws_t, wa_t, wd_t, wc_t, bwc)


def _tc_body(s_ref, a_sum_ref, d_sum_ref, a_ref, c_ref, o_ref):
    y = (jnp.dot(s_ref[...], a_ref[0:D_IN, :])
         + jnp.dot(a_sum_ref[...], a_ref[D_IN:2 * D_IN, :])
         + jnp.dot(d_sum_ref[...], a_ref[2 * D_IN:3 * D_IN, :])
         + c_ref[...])
    y = jnp.where(y >= 0, y, 0.2 * y)
    nrm = jnp.maximum(jnp.sqrt(jnp.sum(y * y, axis=-1, keepdims=True)), 1e-12)
    o_ref[...] = y / nrm


_TC_BLK = 2048


def _tc_dense(m_self, a_sum, d_sum, a, c):
    def whole(shape):
        return pl.BlockSpec(shape, lambda i: tuple(0 for _ in shape))

    row = lambda w: pl.BlockSpec((_TC_BLK, w), lambda i: (i, 0))
    return pl.pallas_call(
        _tc_body,
        grid=(B // _TC_BLK,),
        in_specs=[
            row(D_IN), row(D_IN), row(D_IN),
            whole((3 * D_IN, D_OUT)), whole((1, D_OUT)),
        ],
        out_specs=pl.BlockSpec((_TC_BLK, D_OUT), lambda i: (i, 0)),
        out_shape=jax.ShapeDtypeStruct((B, D_OUT), jnp.float32),
    )(m_self, a_sum, d_sum, a, c)


def kernel(nodes, adj_neighbors, dis_neighbors, table,
           W_agg_adj, b_agg_adj, W_agg_dis, b_agg_dis,
           W_self, W_adj, W_dis, WC, b_WC, bias):
    nodes2 = nodes.astype(jnp.int32).reshape(NW * NSC, SELF_CH)
    adj2 = adj_neighbors.astype(jnp.int32).reshape(NW * NCH, CH * K)
    dis2 = dis_neighbors.astype(jnp.int32).reshape(NW * NCH, CH * K)
    m_self, a_sum, d_sum = _sc_gather(nodes2, adj2, dis2, table)
    a, c = _compose(
        W_agg_adj.T, b_agg_adj.reshape(1, D_IN),
        W_agg_dis.T, b_agg_dis.reshape(1, D_IN),
        W_self.T, W_adj.T, W_dis.T,
        WC.T, (b_WC + bias).reshape(1, D_OUT),
    )
    return _tc_dense(m_self, a_sum, d_sum, a, c)
